# baseline (speedup 1.0000x reference)
import functools

import jax
import jax.numpy as jnp
from jax import lax
from jax.experimental import pallas as pl
from jax.experimental.pallas import tpu as pltpu
from jax.experimental.pallas import tpu_sc as plsc

BATCH = 4096
EMBED_DIM = 64
TOPIC_SIZE = 128
TABLE_ROWS = 100000
PAIR_DIM = 2 * EMBED_DIM
SUP = 16384
HALF = SUP // 2
HBITS = 13
NSUP = -(-TABLE_ROWS // SUP)
PAIR_ROWS = NSUP * HALF

_info = plsc.get_sparse_core_info()
_NC, _NS = _info.num_cores, _info.num_subcores
_NW = _NC * _NS
_BPW = BATCH // _NW


def _tc_repack_body(t1_ref, t2_ref, out_ref):
    out_ref[...] = jnp.concatenate([t1_ref[...], t2_ref[...]], axis=0).T


def _tc_repack(table_t):
    return pl.pallas_call(
        _tc_repack_body,
        grid=(NSUP,),
        in_specs=[
            pl.BlockSpec((EMBED_DIM, HALF), lambda i: (0, 2 * i)),
            pl.BlockSpec((EMBED_DIM, HALF),
                         lambda i: (0, jnp.minimum(2 * i + 1,
                                                   TABLE_ROWS // HALF))),
        ],
        out_specs=pl.BlockSpec((HALF, PAIR_DIM), lambda i: (i, 0)),
        out_shape=jax.ShapeDtypeStruct((PAIR_ROWS, PAIR_DIM), jnp.float32),
    )(table_t, table_t)


def _sc_body(idx_hbm, tab2_hbm, out2_hbm, idx_v, pix_v, rows_v, sem):
    wid = lax.axis_index("s") * _NC + lax.axis_index("c")
    base = wid * _BPW
    pltpu.sync_copy(idx_hbm.at[pl.ds(base, _BPW)], idx_v)
    for j in range(_BPW // 16):
        r = idx_v[pl.ds(j * 16, 16)]
        pix_v[pl.ds(j * 16, 16)] = (
            ((r >> (HBITS + 1)) << HBITS) | (r & (HALF - 1)))
    pltpu.async_copy(tab2_hbm.at[pix_v], rows_v, sem).wait()
    pltpu.sync_copy(rows_v, out2_hbm.at[pl.ds(base, _BPW)])


_sc_gather = functools.partial(
    pl.kernel,
    mesh=plsc.VectorSubcoreMesh(core_axis_name="c", subcore_axis_name="s"),
    out_type=jax.ShapeDtypeStruct((BATCH, PAIR_DIM), jnp.float32),
    scratch_types=[
        pltpu.VMEM((_BPW,), jnp.int32),
        pltpu.VMEM((_BPW,), jnp.int32),
        pltpu.VMEM((_BPW, PAIR_DIM), jnp.float32),
        pltpu.SemaphoreType.DMA,
    ],
)(_sc_body)


def _tc_combine_body(users_ref, items_ref, ctx_ref, topic_ref,
                     eu2_ref, ei2_ref, out_ref):
    pu = ((users_ref[...] >> HBITS) & 1)[:, None] == 1
    pi = ((items_ref[...] >> HBITS) & 1)[:, None] == 1
    eu = jnp.where(pu, eu2_ref[:, EMBED_DIM:], eu2_ref[:, :EMBED_DIM])
    ei = jnp.where(pi, ei2_ref[:, EMBED_DIM:], ei2_ref[:, :EMBED_DIM])
    ctx = jnp.dot(ctx_ref[...], topic_ref[...],
                  preferred_element_type=jnp.float32)
    out_ref[...] = jnp.sum(eu * (ctx + ei), axis=1)


def _tc_combine(users, items, item_ctx, topic_table, eu2, ei2):
    nblk = 2
    bs = BATCH // nblk
    return pl.pallas_call(
        _tc_combine_body,
        grid=(nblk,),
        in_specs=[
            pl.BlockSpec((bs,), lambda i: (i,)),
            pl.BlockSpec((bs,), lambda i: (i,)),
            pl.BlockSpec((bs, TOPIC_SIZE), lambda i: (i, 0)),
            pl.BlockSpec((TOPIC_SIZE, EMBED_DIM), lambda i: (0, 0)),
            pl.BlockSpec((bs, PAIR_DIM), lambda i: (i, 0)),
            pl.BlockSpec((bs, PAIR_DIM), lambda i: (i, 0)),
        ],
        out_specs=pl.BlockSpec((bs,), lambda i: (i,)),
        out_shape=jax.ShapeDtypeStruct((BATCH,), jnp.float32),
    )(users, items, item_ctx, topic_table, eu2, ei2)


@jax.jit
def kernel(users, items, item_ctx, user_table, item_table, topic_table):
    ut2 = _tc_repack(user_table.T)
    eu2 = _sc_gather(users, ut2)
    it2 = _tc_repack(item_table.T)
    ei2 = _sc_gather(items, it2)
    return _tc_combine(users, items, item_ctx, topic_table, eu2, ei2)


# speedup vs baseline: 1.0010x; 1.0010x over previous
import functools

import jax
import jax.numpy as jnp
from jax import lax
from jax.experimental import pallas as pl
from jax.experimental.pallas import tpu as pltpu
from jax.experimental.pallas import tpu_sc as plsc

BATCH = 4096
EMBED_DIM = 64
TOPIC_SIZE = 128
TABLE_ROWS = 100000
PAIR_DIM = 2 * EMBED_DIM
SUP = 16384
HALF = SUP // 2
HBITS = 13
NSUP = -(-TABLE_ROWS // SUP)
PAIR_ROWS = NSUP * HALF

_info = plsc.get_sparse_core_info()
_NC, _NS = _info.num_cores, _info.num_subcores
_NW = _NC * _NS
_BPW = BATCH // _NW


def _tc_repack_body(t1_ref, t2_ref, out_ref):
    out_ref[...] = jnp.concatenate([t1_ref[...], t2_ref[...]], axis=0).T


def _tc_repack(table_t):
    return pl.pallas_call(
        _tc_repack_body,
        grid=(NSUP,),
        in_specs=[
            pl.BlockSpec((EMBED_DIM, HALF), lambda i: (0, 2 * i)),
            pl.BlockSpec((EMBED_DIM, HALF),
                         lambda i: (0, jnp.minimum(2 * i + 1,
                                                   TABLE_ROWS // HALF))),
        ],
        out_specs=pl.BlockSpec((HALF, PAIR_DIM), lambda i: (i, 0)),
        out_shape=jax.ShapeDtypeStruct((PAIR_ROWS, PAIR_DIM), jnp.float32),
    )(table_t, table_t)


def _sc_body(idx_hbm, tab2_hbm, out2_hbm, idx_v, pix_v, rows_v, sem):
    wid = lax.axis_index("s") * _NC + lax.axis_index("c")
    base = wid * _BPW
    pltpu.sync_copy(idx_hbm.at[pl.ds(base, _BPW)], idx_v)
    for j in range(_BPW // 16):
        r = idx_v[pl.ds(j * 16, 16)]
        pix_v[pl.ds(j * 16, 16)] = (
            ((r >> (HBITS + 1)) << HBITS) | (r & (HALF - 1)))
    pltpu.async_copy(tab2_hbm.at[pix_v], rows_v, sem).wait()
    pltpu.sync_copy(rows_v, out2_hbm.at[pl.ds(base, _BPW)])


_sc_gather = functools.partial(
    pl.kernel,
    mesh=plsc.VectorSubcoreMesh(core_axis_name="c", subcore_axis_name="s"),
    out_type=jax.ShapeDtypeStruct((BATCH, PAIR_DIM), jnp.float32),
    scratch_types=[
        pltpu.VMEM((_BPW,), jnp.int32),
        pltpu.VMEM((_BPW,), jnp.int32),
        pltpu.VMEM((_BPW, PAIR_DIM), jnp.float32),
        pltpu.SemaphoreType.DMA,
    ],
)(_sc_body)


def _tc_combine_body(users_ref, items_ref, ctx_ref, topic_ref,
                     eu2_ref, ei2_ref, out_ref):
    pu = ((users_ref[...] >> HBITS) & 1)[:, None] == 1
    pi = ((items_ref[...] >> HBITS) & 1)[:, None] == 1
    eu = jnp.where(pu, eu2_ref[:, EMBED_DIM:], eu2_ref[:, :EMBED_DIM])
    ei = jnp.where(pi, ei2_ref[:, EMBED_DIM:], ei2_ref[:, :EMBED_DIM])
    ctx = jnp.dot(ctx_ref[...], topic_ref[...],
                  preferred_element_type=jnp.float32)
    out_ref[...] = jnp.dot(eu * (ctx + ei), jnp.ones((EMBED_DIM,), jnp.float32),
                           preferred_element_type=jnp.float32)


def _tc_combine(users, items, item_ctx, topic_table, eu2, ei2):
    nblk = 2
    bs = BATCH // nblk
    return pl.pallas_call(
        _tc_combine_body,
        grid=(nblk,),
        in_specs=[
            pl.BlockSpec((bs,), lambda i: (i,)),
            pl.BlockSpec((bs,), lambda i: (i,)),
            pl.BlockSpec((bs, TOPIC_SIZE), lambda i: (i, 0)),
            pl.BlockSpec((TOPIC_SIZE, EMBED_DIM), lambda i: (0, 0)),
            pl.BlockSpec((bs, PAIR_DIM), lambda i: (i, 0)),
            pl.BlockSpec((bs, PAIR_DIM), lambda i: (i, 0)),
        ],
        out_specs=pl.BlockSpec((bs,), lambda i: (i,)),
        out_shape=jax.ShapeDtypeStruct((BATCH,), jnp.float32),
    )(users, items, item_ctx, topic_table, eu2, ei2)


@jax.jit
def kernel(users, items, item_ctx, user_table, item_table, topic_table):
    ut2 = _tc_repack(user_table.T)
    eu2 = _sc_gather(users, ut2)
    it2 = _tc_repack(item_table.T)
    ei2 = _sc_gather(items, it2)
    return _tc_combine(users, items, item_ctx, topic_table, eu2, ei2)
